# baseline (speedup 1.0000x reference)
import functools

import jax
import jax.numpy as jnp
from jax import lax
from jax.experimental import pallas as pl
from jax.experimental.pallas import tpu as pltpu
from jax.experimental.pallas import tpu_sc as plsc

N = 10000
E = 320000
D = 128

NW = 32
K = 64
CPW = 160
NB = 5
NS = 10
EPAD = NW * CPW * K
NCHUNK = EPAD // K
EDGES_PER_W = EPAD // NW
NPAD = 10240
ROWS_PER_SUB = NPAD // 16

_mesh = plsc.VectorSubcoreMesh(core_axis_name="c", subcore_axis_name="s")
_sc_params = pltpu.CompilerParams(needs_layout_passes=False)


@functools.partial(
    pl.kernel,
    out_type=jax.ShapeDtypeStruct((NW, NPAD), jnp.float32),
    mesh=_mesh,
    compiler_params=_sc_params,
    scratch_types=[
        pltpu.VMEM((EDGES_PER_W,), jnp.int32),
        pltpu.VMEM((NPAD,), jnp.float32),
    ],
)
def _deg_kernel(dst_hbm, z_hbm, deg_hbm, dstv, hist):
    cid = lax.axis_index("c")
    sid = lax.axis_index("s")
    wid = sid * 2 + cid
    pltpu.sync_copy(z_hbm, hist)
    pltpu.sync_copy(dst_hbm.at[pl.ds(wid * EDGES_PER_W, EDGES_PER_W)], dstv)
    ones = jnp.full((16,), 1.0, jnp.float32)

    def body(i, carry):
        dv = dstv[pl.ds(i * 16, 16)]
        plsc.addupdate_scatter(hist, [dv], ones)
        return carry

    lax.fori_loop(0, EDGES_PER_W // 16, body, 0)
    pltpu.sync_copy(hist, deg_hbm.at[wid])


@functools.partial(
    pl.kernel,
    out_type=jax.ShapeDtypeStruct((2, NPAD, D), jnp.float32),
    mesh=_mesh,
    compiler_params=_sc_params,
    scratch_types=[
        pltpu.VMEM((2 * NS, K), jnp.int32),
        pltpu.VMEM((NB, K, D), jnp.float32),
        pltpu.VMEM_SHARED((NPAD, D), jnp.float32),
        [pltpu.SemaphoreType.DMA] * NS,
        [pltpu.SemaphoreType.DMA] * NB,
        [pltpu.SemaphoreType.DMA] * NB,
    ],
)
def _scatter_kernel(g_hbm, sd_hbm, z_hbm, out_hbm,
                    sdb, rows, acc, isem, gsem, ssem):
    cid = lax.axis_index("c")
    sid = lax.axis_index("s")
    wid = sid * 2 + cid

    pltpu.sync_copy(z_hbm, rows.at[0])
    for t in range(ROWS_PER_SUB // K):
        pltpu.async_copy(rows.at[0],
                         acc.at[pl.ds(sid * ROWS_PER_SUB + t * K, K)],
                         ssem[0])
    for t in range(ROWS_PER_SUB // K):
        pltpu.make_async_copy(rows.at[0],
                              acc.at[pl.ds(sid * ROWS_PER_SUB + t * K, K)],
                              ssem[0]).wait()
    plsc.subcore_barrier()

    def i_start(c, q):
        pltpu.async_copy(sd_hbm.at[wid * CPW + c], sdb.at[pl.ds(2 * q, 2)],
                         isem[q])

    def i_wait(q):
        pltpu.make_async_copy(sd_hbm.at[0], sdb.at[pl.ds(2 * q, 2)],
                              isem[q]).wait()

    def g_start(c, t, q):
        pltpu.async_copy(g_hbm.at[sdb.at[2 * q]], rows.at[t], gsem[t])

    def g_wait(t):
        pltpu.make_async_copy(g_hbm.at[sdb.at[0]], rows.at[t],
                              gsem[t]).wait()

    def s_start(c, t, q):
        pltpu.async_copy(rows.at[t], acc.at[sdb.at[2 * q + 1]], ssem[t],
                         add=True)

    def s_wait(t):
        pltpu.make_async_copy(rows.at[t], acc.at[sdb.at[1]], ssem[t]).wait()

    def step(c, u, head=0, tail=0):
        t = u % NB
        g_wait(t)
        s_start(c, t, u)
        if not head:
            s_wait((t + 3) % NB)
        if tail < 2:
            i_start(c + 8, (u + 8) % NS)
        if tail < 3:
            i_wait((u + 3) % NS)
            g_start(c + 3, (t + 3) % NB, (u + 3) % NS)

    for c in range(8):
        i_start(c, c)
    for c in range(3):
        i_wait(c)
        g_start(c, c, c)

    for u in range(NS):
        step(u, u, head=(u < 2))

    def body(j, carry):
        c0 = NS * j
        for u in range(NS):
            step(c0 + u, u)
        return carry

    lax.fori_loop(1, CPW // NS - 1, body, 0)

    c0 = CPW - NS
    for u in range(NS):
        c = c0 + u
        tail = 1 if c + 8 <= CPW - 1 else (2 if c + 3 <= CPW - 1 else 3)
        step(c, u, tail=tail)
    s_wait((CPW - 2) % NB)
    s_wait((CPW - 1) % NB)
    plsc.subcore_barrier()
    pltpu.sync_copy(
        acc.at[pl.ds(sid * ROWS_PER_SUB, ROWS_PER_SUB)],
        out_hbm.at[cid, pl.ds(sid * ROWS_PER_SUB, ROWS_PER_SUB)],
    )


BLK = 10000
GRID = N // BLK


def _dinv_from(hist_blk):
    deg = jnp.sum(hist_blk, axis=1, keepdims=True) + 1.0
    return lax.rsqrt(deg)


def _f1_body(h_ref, x_ref, w_ref, g_ref):
    dinv = _dinv_from(h_ref[...])
    h = jnp.dot(x_ref[...], w_ref[...], preferred_element_type=jnp.float32)
    g_ref[...] = h * dinv


def _f2_body(h_ref, s_ref, g1_ref, b_ref, w_ref, g2_ref):
    dinv = _dinv_from(h_ref[...])
    t = (s_ref[0] + s_ref[1] + g1_ref[...]) * dinv + b_ref[...]
    t = jnp.maximum(t, 0.0)
    h = jnp.dot(t, w_ref[...], preferred_element_type=jnp.float32)
    g2_ref[...] = h * dinv


def _f3_body(h_ref, s_ref, g2_ref, b_ref, o_ref):
    dinv = _dinv_from(h_ref[...])
    o_ref[...] = (s_ref[0] + s_ref[1] + g2_ref[...]) * dinv + b_ref[...]


_row_spec = pl.BlockSpec((BLK, D), lambda i: (i, 0))
_s_spec = pl.BlockSpec((2, BLK, D), lambda i: (0, i, 0))
_hist_spec = pl.BlockSpec((BLK, NW), lambda i: (i, 0))
_w_spec = pl.BlockSpec((D, D), lambda i: (0, 0))
_b_spec = pl.BlockSpec((1, D), lambda i: (0, 0))
_out_row = jax.ShapeDtypeStruct((N, D), jnp.float32)

_fuse1 = pl.pallas_call(
    _f1_body,
    grid=(GRID,),
    in_specs=[_hist_spec, _row_spec, _w_spec],
    out_specs=_row_spec,
    out_shape=_out_row,
)

_fuse2 = pl.pallas_call(
    _f2_body,
    grid=(GRID,),
    in_specs=[_hist_spec, _s_spec, _row_spec, _b_spec, _w_spec],
    out_specs=_row_spec,
    out_shape=_out_row,
)

_fuse3 = pl.pallas_call(
    _f3_body,
    grid=(GRID,),
    in_specs=[_hist_spec, _s_spec, _row_spec, _b_spec],
    out_specs=_row_spec,
    out_shape=_out_row,
)


def kernel(x, edge_index, W1, b1, W2, b2):
    npad_e = EPAD - E
    pad_src = (jnp.arange(npad_e, dtype=jnp.int32) * 13) % N
    pad_dst = N + (jnp.arange(npad_e, dtype=jnp.int32) % (NPAD - N))
    src = jnp.concatenate([edge_index[0], pad_src])
    dst = jnp.concatenate([edge_index[1], pad_dst])
    sd = jnp.stack([src.reshape(NCHUNK, K), dst.reshape(NCHUNK, K)], axis=1)
    z1 = jnp.zeros((NPAD,), jnp.float32)
    z2 = jnp.zeros((K, D), jnp.float32)

    hists = _deg_kernel(dst, z1)
    hT = jnp.transpose(hists)[:N]

    g1 = _fuse1(hT, x, W1)
    s1 = _scatter_kernel(g1, sd, z2)
    g2 = _fuse2(hT, s1, g1, b1.reshape(1, D), W2)
    s2 = _scatter_kernel(g2, sd, z2)
    out = _fuse3(hT, s2, g2, b2.reshape(1, D))
    return out


# speedup vs baseline: 1.0396x; 1.0396x over previous
import functools

import jax
import jax.numpy as jnp
from jax import lax
from jax.experimental import pallas as pl
from jax.experimental.pallas import tpu as pltpu
from jax.experimental.pallas import tpu_sc as plsc

N = 10000
E = 320000
D = 128

NW = 32
K = 64
CPW = 160
NB = 5
NS = 10
EPAD = NW * CPW * K
NCHUNK = EPAD // K
EDGES_PER_W = EPAD // NW
NPAD = 10240
ROWS_PER_SUB = NPAD // 16

_mesh = plsc.VectorSubcoreMesh(core_axis_name="c", subcore_axis_name="s")
_sc_params = pltpu.CompilerParams(needs_layout_passes=False)


@functools.partial(
    pl.kernel,
    out_type=jax.ShapeDtypeStruct((NW, NPAD), jnp.float32),
    mesh=_mesh,
    compiler_params=_sc_params,
    scratch_types=[
        pltpu.VMEM((EDGES_PER_W,), jnp.int32),
        pltpu.VMEM((NPAD,), jnp.float32),
    ],
)
def _deg_kernel(dst_hbm, z_hbm, deg_hbm, dstv, hist):
    cid = lax.axis_index("c")
    sid = lax.axis_index("s")
    wid = sid * 2 + cid
    pltpu.sync_copy(z_hbm, hist)
    pltpu.sync_copy(dst_hbm.at[pl.ds(wid * EDGES_PER_W, EDGES_PER_W)], dstv)
    ones = jnp.full((16,), 1.0, jnp.float32)

    def body(i, carry):
        dv = dstv[pl.ds(i * 16, 16)]
        plsc.addupdate_scatter(hist, [dv], ones)
        return carry

    lax.fori_loop(0, EDGES_PER_W // 16, body, 0)
    pltpu.sync_copy(hist, deg_hbm.at[wid])


@functools.partial(
    pl.kernel,
    out_type=jax.ShapeDtypeStruct((2, NPAD, D), jnp.float32),
    mesh=_mesh,
    compiler_params=_sc_params,
    scratch_types=[
        pltpu.VMEM((2 * NS, K), jnp.int32),
        pltpu.VMEM((NB, K, D), jnp.float32),
        pltpu.VMEM_SHARED((NPAD, D), jnp.float32),
        [pltpu.SemaphoreType.DMA] * NS,
        [pltpu.SemaphoreType.DMA] * NB,
        [pltpu.SemaphoreType.DMA] * NB,
    ],
)
def _scatter_kernel(g_hbm, sd_hbm, z_hbm, out_hbm,
                    sdb, rows, acc, isem, gsem, ssem):
    cid = lax.axis_index("c")
    sid = lax.axis_index("s")
    wid = sid * 2 + cid

    pltpu.sync_copy(z_hbm, rows.at[0])
    for t in range(ROWS_PER_SUB // K):
        pltpu.async_copy(rows.at[0],
                         acc.at[pl.ds(sid * ROWS_PER_SUB + t * K, K)],
                         ssem[0])
    for t in range(ROWS_PER_SUB // K):
        pltpu.make_async_copy(rows.at[0],
                              acc.at[pl.ds(sid * ROWS_PER_SUB + t * K, K)],
                              ssem[0]).wait()
    plsc.subcore_barrier()

    def i_start(c, q):
        pltpu.async_copy(sd_hbm.at[wid * CPW + c], sdb.at[pl.ds(2 * q, 2)],
                         isem[q])

    def i_wait(q):
        pltpu.make_async_copy(sd_hbm.at[0], sdb.at[pl.ds(2 * q, 2)],
                              isem[q]).wait()

    def g_start(c, t, q):
        pltpu.async_copy(g_hbm.at[sdb.at[2 * q]], rows.at[t], gsem[t])

    def g_wait(t):
        pltpu.make_async_copy(g_hbm.at[sdb.at[0]], rows.at[t],
                              gsem[t]).wait()

    def s_start(c, t, q):
        pltpu.async_copy(rows.at[t], acc.at[sdb.at[2 * q + 1]], ssem[t],
                         add=True)

    def s_wait(t):
        pltpu.make_async_copy(rows.at[t], acc.at[sdb.at[1]], ssem[t]).wait()

    def step(c, u, head=0, tail=0):
        t = u % NB
        g_wait(t)
        s_start(c, t, u)
        if not head:
            s_wait((t + 4) % NB)
        if tail < 2:
            i_start(c + 9, (u + 9) % NS)
        if tail < 3:
            i_wait((u + 4) % NS)
            g_start(c + 4, (t + 4) % NB, (u + 4) % NS)

    for c in range(9):
        i_start(c, c)
    for c in range(4):
        i_wait(c)
        g_start(c, c, c)

    for u in range(NS):
        step(u, u, head=(u < 1))

    def body(j, carry):
        c0 = NS * j
        for u in range(NS):
            step(c0 + u, u)
        return carry

    lax.fori_loop(1, CPW // NS - 1, body, 0)

    c0 = CPW - NS
    for u in range(NS):
        c = c0 + u
        tail = 1 if c + 9 <= CPW - 1 else (2 if c + 4 <= CPW - 1 else 3)
        step(c, u, tail=tail)
    s_wait((CPW - 1) % NB)
    plsc.subcore_barrier()
    pltpu.sync_copy(
        acc.at[pl.ds(sid * ROWS_PER_SUB, ROWS_PER_SUB)],
        out_hbm.at[cid, pl.ds(sid * ROWS_PER_SUB, ROWS_PER_SUB)],
    )


BLK = 5000
GRID = N // BLK


def _dinv_from(hist_blk):
    deg = jnp.sum(hist_blk, axis=1, keepdims=True) + 1.0
    return lax.rsqrt(deg)


def _f1_body(h_ref, x_ref, w_ref, g_ref):
    dinv = _dinv_from(h_ref[...])
    h = jnp.dot(x_ref[...], w_ref[...], preferred_element_type=jnp.float32)
    g_ref[...] = h * dinv


def _f2_body(h_ref, s_ref, g1_ref, b_ref, w_ref, g2_ref):
    dinv = _dinv_from(h_ref[...])
    t = (s_ref[0] + s_ref[1] + g1_ref[...]) * dinv + b_ref[...]
    t = jnp.maximum(t, 0.0)
    h = jnp.dot(t, w_ref[...], preferred_element_type=jnp.float32)
    g2_ref[...] = h * dinv


def _f3_body(h_ref, s_ref, g2_ref, b_ref, o_ref):
    dinv = _dinv_from(h_ref[...])
    o_ref[...] = (s_ref[0] + s_ref[1] + g2_ref[...]) * dinv + b_ref[...]


_row_spec = pl.BlockSpec((BLK, D), lambda i: (i, 0))
_s_spec = pl.BlockSpec((2, BLK, D), lambda i: (0, i, 0))
_hist_spec = pl.BlockSpec((BLK, NW), lambda i: (i, 0))
_w_spec = pl.BlockSpec((D, D), lambda i: (0, 0))
_b_spec = pl.BlockSpec((1, D), lambda i: (0, 0))
_out_row = jax.ShapeDtypeStruct((N, D), jnp.float32)

_fuse1 = pl.pallas_call(
    _f1_body,
    grid=(GRID,),
    in_specs=[_hist_spec, _row_spec, _w_spec],
    out_specs=_row_spec,
    out_shape=_out_row,
)

_fuse2 = pl.pallas_call(
    _f2_body,
    grid=(GRID,),
    in_specs=[_hist_spec, _s_spec, _row_spec, _b_spec, _w_spec],
    out_specs=_row_spec,
    out_shape=_out_row,
)

_fuse3 = pl.pallas_call(
    _f3_body,
    grid=(GRID,),
    in_specs=[_hist_spec, _s_spec, _row_spec, _b_spec],
    out_specs=_row_spec,
    out_shape=_out_row,
)


def kernel(x, edge_index, W1, b1, W2, b2):
    npad_e = EPAD - E
    pad_src = (jnp.arange(npad_e, dtype=jnp.int32) * 13) % N
    pad_dst = N + (jnp.arange(npad_e, dtype=jnp.int32) % (NPAD - N))
    src = jnp.concatenate([edge_index[0], pad_src])
    dst = jnp.concatenate([edge_index[1], pad_dst])
    sd = jnp.stack([src.reshape(NCHUNK, K), dst.reshape(NCHUNK, K)], axis=1)
    z1 = jnp.zeros((NPAD,), jnp.float32)
    z2 = jnp.zeros((K, D), jnp.float32)

    hists = _deg_kernel(dst, z1)
    hT = jnp.transpose(hists)[:N]

    g1 = _fuse1(hT, x, W1)
    s1 = _scatter_kernel(g1, sd, z2)
    g2 = _fuse2(hT, s1, g1, b1.reshape(1, D), W2)
    s2 = _scatter_kernel(g2, sd, z2)
    out = _fuse3(hT, s2, g2, b2.reshape(1, D))
    return out
